# baseline (speedup 1.0000x reference)
import functools

import jax
import jax.numpy as jnp
from jax import lax
from jax.experimental import pallas as pl
from jax.experimental.pallas import tpu as pltpu
from jax.experimental.pallas import tpu_sc as plsc

NC = 2
NS = 16
NW = NC * NS
L = 16


def _premul_nodes(hidden, W1):
    N, D = hidden.shape
    R = 2000
    assert N % R == 0

    def body(h_ref, w1_ref, u_ref, v_ref):
        h = h_ref[...]
        w1 = w1_ref[...]
        u_ref[...] = jnp.dot(
            h, w1[0:D], preferred_element_type=jnp.float32
        ).astype(jnp.bfloat16)
        v_ref[...] = jnp.dot(
            h, w1[D:2 * D], preferred_element_type=jnp.float32
        ).astype(jnp.bfloat16)

    W1K = W1.shape[0]
    return pl.pallas_call(
        body,
        grid=(N // R,),
        in_specs=[
            pl.BlockSpec((R, D), lambda i: (i, 0)),
            pl.BlockSpec((W1K, D), lambda i: (0, 0)),
        ],
        out_specs=[pl.BlockSpec((R, D), lambda i: (i, 0))] * 2,
        out_shape=[jax.ShapeDtypeStruct((N, D), jnp.bfloat16)] * 2,
    )(hidden, W1)


def _premul_side(times_table, interval_table, connection_table, W1, b1, K):
    NT, TH = times_table.shape
    NI, IH = interval_table.shape
    NCN = connection_table.shape[0]
    D = W1.shape[1]

    def body(tt, it, ct, w1_ref, b1r, tab):
        w1 = w1_ref[...]
        wt = w1[2 * D:2 * D + TH]
        wi = w1[2 * D + TH:2 * D + TH + IH]
        wc = w1[2 * D + TH + IH:]
        tab[0:NT] = (jnp.dot(tt[...], wt, preferred_element_type=jnp.float32)
                     + b1r[...]).astype(jnp.bfloat16)
        tab[NT:NT + NI] = jnp.dot(
            it[...], wi, preferred_element_type=jnp.float32
        ).astype(jnp.bfloat16)
        tab[NT + NI:NT + NI + NCN] = jnp.dot(
            ct[...], wc, preferred_element_type=jnp.float32
        ).astype(jnp.bfloat16)
        tab[NT + NI + NCN:K] = jnp.zeros(
            (K - NT - NI - NCN, D), jnp.bfloat16)

    return pl.pallas_call(
        body,
        out_shape=jax.ShapeDtypeStruct((K, D), jnp.bfloat16),
    )(times_table, interval_table, connection_table, W1, b1.reshape(1, D))


def _premul_ti(tpart, ipart):
    NT, D = tpart.shape
    NI = ipart.shape[0]
    TB = 8
    assert NT % TB == 0

    def body(t_ref, i_ref, ti_ref):
        t_rows = t_ref[...]
        i_rows = i_ref[...]
        ti_ref[...] = (t_rows[:, None, :] + i_rows[None, :, :]).reshape(
            TB * NI, D)

    return pl.pallas_call(
        body,
        grid=(NT // TB,),
        in_specs=[
            pl.BlockSpec((TB, D), lambda i: (i, 0)),
            pl.BlockSpec((NI, D), lambda i: (0, 0)),
        ],
        out_specs=pl.BlockSpec((TB * NI, D), lambda i: (i, 0)),
        out_shape=jax.ShapeDtypeStruct((NT * NI, D), jnp.bfloat16),
    )(tpart, ipart)


def _make_sc_score(B, D, K, E, NT, NI):
    per_w = B // NW
    chunks = per_w // E
    assert chunks % 2 == 1 and E % L == 0
    W = 2 * L
    mesh = plsc.VectorSubcoreMesh(
        core_axis_name="c", subcore_axis_name="s", num_cores=NC, num_subcores=NS)

    @functools.partial(
        pl.kernel,
        out_type=jax.ShapeDtypeStruct((B,), jnp.float32),
        mesh=mesh,
        compiler_params=pltpu.CompilerParams(
            needs_layout_passes=False, use_tc_tiling_on_sc=False),
        scratch_types=[
            pltpu.VMEM((K, D), jnp.bfloat16),
            pltpu.VMEM((D,), jnp.bfloat16),
            pltpu.VMEM((L,), jnp.float32),
            pltpu.VMEM((L, L), jnp.float32),
            pltpu.VMEM((5, E), jnp.int32),
            pltpu.VMEM((5, E), jnp.int32),
            pltpu.VMEM((E,), jnp.int32),
            pltpu.VMEM((E,), jnp.int32),
            pltpu.VMEM((E, D), jnp.bfloat16),
            pltpu.VMEM((E, D), jnp.bfloat16),
            pltpu.VMEM((E, D), jnp.bfloat16),
            pltpu.VMEM((E, D), jnp.bfloat16),
            pltpu.VMEM((E, D), jnp.bfloat16),
            pltpu.VMEM((E, D), jnp.bfloat16),
            pltpu.VMEM((E,), jnp.float32),
            pltpu.VMEM((E,), jnp.float32),
            pltpu.SemaphoreType.DMA,
            pltpu.SemaphoreType.DMA,
            pltpu.SemaphoreType.DMA,
            pltpu.SemaphoreType.DMA,
            pltpu.SemaphoreType.DMA,
            pltpu.SemaphoreType.DMA,
        ],
    )
    def sc_score(u_hbm, v_hbm, ti_hbm, side_hbm, w2_hbm, b2_hbm,
                 src_hbm, dst_hbm, t_hbm, i_hbm, c_hbm, s_hbm,
                 side_v, w2_v, b2_v, mat_v, idx0, idx1, pb0, pb1,
                 u0, u1, v0, v1, ti0, ti1, s0, s1,
                 semi0, semi1, semg0, semg1, semo0, semo1):
        wid = lax.axis_index("s") * NC + lax.axis_index("c")
        base = wid * per_w
        pltpu.sync_copy(side_hbm, side_v)
        pltpu.sync_copy(w2_hbm, w2_v)
        pltpu.sync_copy(b2_hbm, b2_v)
        w2b = [w2_v[pl.ds(kk * W, W)] for kk in range(D // W)]
        lanes = lax.iota(jnp.int32, L)
        streams = (src_hbm, dst_hbm, t_hbm, i_hbm, c_hbm)

        def issue(k, idx_p, pb_p, u_p, v_p, ti_p, semi, semg):
            cbase = base + k * E
            for r in range(5):
                pltpu.async_copy(
                    streams[r].at[pl.ds(cbase, E)], idx_p.at[r], semi)
            for r in range(5):
                pltpu.make_async_copy(
                    streams[r].at[pl.ds(0, E)], idx_p.at[r], semi).wait()
            for g in range(E // L):
                sl = pl.ds(g * L, L)
                pb_p[sl] = idx_p[2, sl] * NI + idx_p[3, sl]
            pltpu.async_copy(u_hbm.at[idx_p.at[0]], u_p, semg)
            pltpu.async_copy(v_hbm.at[idx_p.at[1]], v_p, semg)
            pltpu.async_copy(ti_hbm.at[pb_p], ti_p, semg)

        def drain_g(u_p, v_p, ti_p, semg):
            pltpu.make_async_copy(u_hbm.at[pl.ds(0, E)], u_p, semg).wait()
            pltpu.make_async_copy(v_hbm.at[pl.ds(0, E)], v_p, semg).wait()
            pltpu.make_async_copy(u_hbm.at[pl.ds(0, E)], ti_p, semg).wait()

        def drain_o(s_p, semo):
            pltpu.make_async_copy(s_hbm.at[pl.ds(0, E)], s_p, semo).wait()

        def score_chunk(idx_p, u_p, v_p, ti_p, s_p):
            def gbody(g, carry):
                cvec = idx_p[4, pl.ds(g * L, L)] + (NT + NI)
                for j in range(L):
                    e = g * L + j
                    c = cvec[j]
                    parts = []
                    for kk in range(D // W):
                        s = pl.ds(kk * W, W)
                        a = u_p[e, s] + v_p[e, s]
                        b = ti_p[e, s] + side_v[c, s]
                        z = a + b
                        p = jnp.maximum(z, jnp.bfloat16(0)) * w2b[kk]
                        pa, pb = plsc.unpack(
                            p, format=plsc.PackFormat.INTERLEAVED,
                            preferred_element_type=jnp.float32)
                        parts.append(pa + pb)
                    mat_v[j, :] = (parts[0] + parts[1]) + (parts[2] + parts[3])
                acc = b2_v[...] + plsc.load_gather(
                    mat_v, [lanes, jnp.full((L,), 0, jnp.int32)])
                for col in range(1, L):
                    acc = acc + plsc.load_gather(
                        mat_v, [lanes, jnp.full((L,), col, jnp.int32)])
                s_p[pl.ds(g * L, L)] = acc
                return carry
            lax.fori_loop(0, E // L, gbody, 0)

        issue(0, idx0, pb0, u0, v0, ti0, semi0, semg0)

        def pair_body(m, carry):
            a = 2 * m
            issue(a + 1, idx1, pb1, u1, v1, ti1, semi1, semg1)

            @pl.when(m > 0)
            def _():
                drain_o(s0, semo0)

            drain_g(u0, v0, ti0, semg0)
            score_chunk(idx0, u0, v0, ti0, s0)
            pltpu.async_copy(s0, s_hbm.at[pl.ds(base + a * E, E)], semo0)
            issue(a + 2, idx0, pb0, u0, v0, ti0, semi0, semg0)

            @pl.when(m > 0)
            def _():
                drain_o(s1, semo1)

            drain_g(u1, v1, ti1, semg1)
            score_chunk(idx1, u1, v1, ti1, s1)
            pltpu.async_copy(s1, s_hbm.at[pl.ds(base + (a + 1) * E, E)], semo1)
            return carry

        lax.fori_loop(0, (chunks - 1) // 2, pair_body, 0)
        drain_o(s0, semo0)
        drain_g(u0, v0, ti0, semg0)
        score_chunk(idx0, u0, v0, ti0, s0)
        pltpu.async_copy(s0, s_hbm.at[pl.ds(base + (chunks - 1) * E, E)], semo0)
        drain_o(s0, semo0)
        drain_o(s1, semo1)

    return sc_score


def kernel(hidden, times_table, interval_table, connection_table, W1, b1, W2,
           b2, source, destination, times, intervals, connection_types):
    N, D = hidden.shape
    TH = times_table.shape[1]
    IH = interval_table.shape[1]
    B = source.shape[0]
    NT = times_table.shape[0]
    NI = interval_table.shape[0]
    NCN = connection_table.shape[0]
    E = 80
    assert B % (NW * E) == 0

    U, V = _premul_nodes(hidden, W1)
    K = NT + NI + NCN + ((-(NT + NI + NCN)) % 8)
    sidetab = _premul_side(times_table, interval_table, connection_table,
                           W1, b1, K)
    NTP = NT + ((-NT) % 8)
    ti_tab = _premul_ti(sidetab[:NTP], sidetab[NT:NT + NI])

    sc_score = _make_sc_score(B, D, K, E, NT, NI)
    scores = sc_score(U, V, ti_tab, sidetab,
                      W2.reshape(D).astype(jnp.bfloat16),
                      jnp.broadcast_to(b2, (L,)),
                      source, destination, times, intervals, connection_types)
    return scores


# speedup vs baseline: 1.1426x; 1.1426x over previous
import functools

import jax
import jax.numpy as jnp
from jax import lax
from jax.experimental import pallas as pl
from jax.experimental.pallas import tpu as pltpu
from jax.experimental.pallas import tpu_sc as plsc

NC = 2
NS = 16
NW = NC * NS
L = 16


def _premul_nodes(hidden, W1):
    N, D = hidden.shape
    R = 2000
    assert N % R == 0

    def body(h_ref, w1_ref, u_ref, v_ref):
        h = h_ref[...]
        w1 = w1_ref[...]
        u_ref[...] = jnp.dot(
            h, w1[0:D], preferred_element_type=jnp.float32
        ).astype(jnp.bfloat16)
        v_ref[...] = jnp.dot(
            h, w1[D:2 * D], preferred_element_type=jnp.float32
        ).astype(jnp.bfloat16)

    W1K = W1.shape[0]
    return pl.pallas_call(
        body,
        grid=(N // R,),
        in_specs=[
            pl.BlockSpec((R, D), lambda i: (i, 0)),
            pl.BlockSpec((W1K, D), lambda i: (0, 0)),
        ],
        out_specs=[pl.BlockSpec((R, D), lambda i: (i, 0))] * 2,
        out_shape=[jax.ShapeDtypeStruct((N, D), jnp.bfloat16)] * 2,
    )(hidden, W1)


def _premul_side(times_table, interval_table, connection_table, W1, b1, K):
    NT, TH = times_table.shape
    NI, IH = interval_table.shape
    NCN = connection_table.shape[0]
    D = W1.shape[1]

    def body(tt, it, ct, w1_ref, b1r, tab):
        w1 = w1_ref[...]
        wt = w1[2 * D:2 * D + TH]
        wi = w1[2 * D + TH:2 * D + TH + IH]
        wc = w1[2 * D + TH + IH:]
        tab[0:NT] = (jnp.dot(tt[...], wt, preferred_element_type=jnp.float32)
                     + b1r[...]).astype(jnp.bfloat16)
        tab[NT:NT + NI] = jnp.dot(
            it[...], wi, preferred_element_type=jnp.float32
        ).astype(jnp.bfloat16)
        tab[NT + NI:NT + NI + NCN] = jnp.dot(
            ct[...], wc, preferred_element_type=jnp.float32
        ).astype(jnp.bfloat16)
        tab[NT + NI + NCN:K] = jnp.zeros(
            (K - NT - NI - NCN, D), jnp.bfloat16)

    return pl.pallas_call(
        body,
        out_shape=jax.ShapeDtypeStruct((K, D), jnp.bfloat16),
    )(times_table, interval_table, connection_table, W1, b1.reshape(1, D))


def _make_sc_score(B, D, K, E, NT, NI):
    per_w = B // NW
    chunks = per_w // E
    assert chunks % 2 == 1 and E % L == 0
    W = 2 * L
    mesh = plsc.VectorSubcoreMesh(
        core_axis_name="c", subcore_axis_name="s", num_cores=NC, num_subcores=NS)

    @functools.partial(
        pl.kernel,
        out_type=jax.ShapeDtypeStruct((B,), jnp.float32),
        mesh=mesh,
        compiler_params=pltpu.CompilerParams(
            needs_layout_passes=False, use_tc_tiling_on_sc=False),
        scratch_types=[
            pltpu.VMEM((K, D), jnp.bfloat16),
            pltpu.VMEM((D,), jnp.bfloat16),
            pltpu.VMEM((L,), jnp.float32),
            pltpu.VMEM((L, L), jnp.float32),
            pltpu.VMEM((5, E), jnp.int32),
            pltpu.VMEM((5, E), jnp.int32),
            pltpu.VMEM((5, E), jnp.int32),
            pltpu.VMEM((5, E), jnp.int32),
            pltpu.VMEM((E, D), jnp.bfloat16),
            pltpu.VMEM((E, D), jnp.bfloat16),
            pltpu.VMEM((E, D), jnp.bfloat16),
            pltpu.VMEM((E, D), jnp.bfloat16),
            pltpu.VMEM((E,), jnp.float32),
            pltpu.VMEM((E,), jnp.float32),
            pltpu.SemaphoreType.DMA,
            pltpu.SemaphoreType.DMA,
            pltpu.SemaphoreType.DMA,
            pltpu.SemaphoreType.DMA,
            pltpu.SemaphoreType.DMA,
            pltpu.SemaphoreType.DMA,
            pltpu.SemaphoreType.DMA,
            pltpu.SemaphoreType.DMA,
        ],
    )
    def sc_score(u_hbm, v_hbm, side_hbm, w2_hbm, b2_hbm,
                 src_hbm, dst_hbm, t_hbm, i_hbm, c_hbm, s_hbm,
                 side_v, w2_v, b2_v, mat_v, idxA0, idxA1, idxB0, idxB1,
                 u0, u1, v0, v1, s0, s1,
                 semiA0, semiA1, semiB0, semiB1, semg0, semg1, semo0, semo1):
        wid = lax.axis_index("s") * NC + lax.axis_index("c")
        base = wid * per_w
        pltpu.sync_copy(side_hbm, side_v)
        pltpu.sync_copy(w2_hbm, w2_v)
        pltpu.sync_copy(b2_hbm, b2_v)
        w2b = [w2_v[pl.ds(kk * W, W)] for kk in range(D // W)]
        lanes = lax.iota(jnp.int32, L)
        streams = (src_hbm, dst_hbm, t_hbm, i_hbm, c_hbm)

        def start_idx(k, idx_p, semi):
            cbase = base + k * E
            for r in range(5):
                pltpu.async_copy(
                    streams[r].at[pl.ds(cbase, E)], idx_p.at[r], semi)

        def fire_g(idx_p, semi, u_p, v_p, semg):
            for r in range(5):
                pltpu.make_async_copy(
                    streams[r].at[pl.ds(0, E)], idx_p.at[r], semi).wait()
            pltpu.async_copy(u_hbm.at[idx_p.at[0]], u_p, semg)
            pltpu.async_copy(v_hbm.at[idx_p.at[1]], v_p, semg)

        def drain_g(u_p, v_p, semg):
            pltpu.make_async_copy(u_hbm.at[pl.ds(0, E)], u_p, semg).wait()
            pltpu.make_async_copy(v_hbm.at[pl.ds(0, E)], v_p, semg).wait()

        def drain_o(s_p, semo):
            pltpu.make_async_copy(s_hbm.at[pl.ds(0, E)], s_p, semo).wait()

        def score_chunk(idx_p, u_p, v_p, s_p):
            def gbody(g, carry):
                tvec = idx_p[2, pl.ds(g * L, L)]
                ivec = idx_p[3, pl.ds(g * L, L)] + NT
                cvec = idx_p[4, pl.ds(g * L, L)] + (NT + NI)
                for j in range(L):
                    e = g * L + j
                    t = tvec[j]
                    i = ivec[j]
                    c = cvec[j]
                    parts = []
                    for kk in range(D // W):
                        s = pl.ds(kk * W, W)
                        a = u_p[e, s] + v_p[e, s]
                        b = side_v[t, s] + side_v[i, s]
                        z = (a + b) + side_v[c, s]
                        p = jnp.maximum(z, jnp.bfloat16(0)) * w2b[kk]
                        pa, pb = plsc.unpack(
                            p, format=plsc.PackFormat.INTERLEAVED,
                            preferred_element_type=jnp.float32)
                        parts.append(pa + pb)
                    mat_v[j, :] = (parts[0] + parts[1]) + (parts[2] + parts[3])
                acc = b2_v[...] + plsc.load_gather(
                    mat_v, [lanes, jnp.full((L,), 0, jnp.int32)])
                for col in range(1, L):
                    acc = acc + plsc.load_gather(
                        mat_v, [lanes, jnp.full((L,), col, jnp.int32)])
                s_p[pl.ds(g * L, L)] = acc
                return carry
            lax.fori_loop(0, E // L, gbody, 0)

        def out(s_p, k, semo):
            pltpu.async_copy(s_p, s_hbm.at[pl.ds(base + k * E, E)], semo)

        QM = (chunks - 1) // 4
        start_idx(0, idxA0, semiA0)
        start_idx(1, idxA1, semiA1)
        start_idx(2, idxB0, semiB0)
        fire_g(idxA0, semiA0, u0, v0, semg0)
        fire_g(idxA1, semiA1, u1, v1, semg1)

        def quad_body(m, carry):
            q = 4 * m
            start_idx(q + 3, idxB1, semiB1)

            @pl.when(m > 0)
            def _():
                drain_o(s0, semo0)

            drain_g(u0, v0, semg0)
            score_chunk(idxA0, u0, v0, s0)
            out(s0, q, semo0)
            fire_g(idxB0, semiB0, u0, v0, semg0)
            start_idx(q + 4, idxA0, semiA0)

            @pl.when(m > 0)
            def _():
                drain_o(s1, semo1)

            drain_g(u1, v1, semg1)
            score_chunk(idxA1, u1, v1, s1)
            out(s1, q + 1, semo1)
            fire_g(idxB1, semiB1, u1, v1, semg1)

            @pl.when(m < QM - 1)
            def _():
                start_idx(q + 5, idxA1, semiA1)

            drain_o(s0, semo0)
            drain_g(u0, v0, semg0)
            score_chunk(idxB0, u0, v0, s0)
            out(s0, q + 2, semo0)
            fire_g(idxA0, semiA0, u0, v0, semg0)

            @pl.when(m < QM - 1)
            def _():
                start_idx(q + 6, idxB0, semiB0)

            drain_o(s1, semo1)
            drain_g(u1, v1, semg1)
            score_chunk(idxB1, u1, v1, s1)
            out(s1, q + 3, semo1)

            @pl.when(m < QM - 1)
            def _():
                fire_g(idxA1, semiA1, u1, v1, semg1)

            return carry

        assert chunks % 4 == 1
        lax.fori_loop(0, QM, quad_body, 0)
        drain_o(s0, semo0)
        drain_g(u0, v0, semg0)
        score_chunk(idxA0, u0, v0, s0)
        out(s0, chunks - 1, semo0)
        drain_o(s0, semo0)
        drain_o(s1, semo1)

    return sc_score


def kernel(hidden, times_table, interval_table, connection_table, W1, b1, W2,
           b2, source, destination, times, intervals, connection_types):
    N, D = hidden.shape
    TH = times_table.shape[1]
    IH = interval_table.shape[1]
    B = source.shape[0]
    NT = times_table.shape[0]
    NI = interval_table.shape[0]
    NCN = connection_table.shape[0]
    E = 80
    assert B % (NW * E) == 0

    U, V = _premul_nodes(hidden, W1)
    K = NT + NI + NCN + ((-(NT + NI + NCN)) % 8)
    sidetab = _premul_side(times_table, interval_table, connection_table,
                           W1, b1, K)
    sc_score = _make_sc_score(B, D, K, E, NT, NI)
    scores = sc_score(U, V, sidetab,
                      W2.reshape(D).astype(jnp.bfloat16),
                      jnp.broadcast_to(b2, (L,)),
                      source, destination, times, intervals, connection_types)
    return scores


# speedup vs baseline: 1.2060x; 1.0555x over previous
import functools

import jax
import jax.numpy as jnp
from jax import lax
from jax.experimental import pallas as pl
from jax.experimental.pallas import tpu as pltpu
from jax.experimental.pallas import tpu_sc as plsc

NC = 2
NS = 16
NW = NC * NS
L = 16


def _premul_nodes(hidden, W1):
    N, D = hidden.shape
    R = 4000
    assert N % R == 0

    def body(h_ref, w1_ref, u_ref, v_ref):
        h = h_ref[...]
        w1 = w1_ref[...]
        u_ref[...] = jnp.dot(
            h, w1[0:D], preferred_element_type=jnp.float32
        ).astype(jnp.bfloat16)
        v_ref[...] = jnp.dot(
            h, w1[D:2 * D], preferred_element_type=jnp.float32
        ).astype(jnp.bfloat16)

    W1K = W1.shape[0]
    return pl.pallas_call(
        body,
        grid=(N // R,),
        in_specs=[
            pl.BlockSpec((R, D), lambda i: (i, 0)),
            pl.BlockSpec((W1K, D), lambda i: (0, 0)),
        ],
        out_specs=[pl.BlockSpec((R, D), lambda i: (i, 0))] * 2,
        out_shape=[jax.ShapeDtypeStruct((N, D), jnp.bfloat16)] * 2,
    )(hidden, W1)


def _premul_side(times_table, interval_table, connection_table, W1, b1, K):
    NT, TH = times_table.shape
    NI, IH = interval_table.shape
    NCN = connection_table.shape[0]
    D = W1.shape[1]

    def body(tt, it, ct, w1_ref, b1r, tab):
        w1 = w1_ref[...]
        wt = w1[2 * D:2 * D + TH]
        wi = w1[2 * D + TH:2 * D + TH + IH]
        wc = w1[2 * D + TH + IH:]
        tab[0:NT] = (jnp.dot(tt[...], wt, preferred_element_type=jnp.float32)
                     + b1r[...]).astype(jnp.bfloat16)
        tab[NT:NT + NI] = jnp.dot(
            it[...], wi, preferred_element_type=jnp.float32
        ).astype(jnp.bfloat16)
        tab[NT + NI:NT + NI + NCN] = jnp.dot(
            ct[...], wc, preferred_element_type=jnp.float32
        ).astype(jnp.bfloat16)
        tab[NT + NI + NCN:K] = jnp.zeros(
            (K - NT - NI - NCN, D), jnp.bfloat16)

    return pl.pallas_call(
        body,
        out_shape=jax.ShapeDtypeStruct((K, D), jnp.bfloat16),
    )(times_table, interval_table, connection_table, W1, b1.reshape(1, D))


def _make_sc_score(B, D, K, E, NT, NI):
    per_w = B // NW
    chunks = per_w // E
    assert chunks % 2 == 1 and E % L == 0
    W = 2 * L
    mesh = plsc.VectorSubcoreMesh(
        core_axis_name="c", subcore_axis_name="s", num_cores=NC, num_subcores=NS)

    @functools.partial(
        pl.kernel,
        out_type=jax.ShapeDtypeStruct((B,), jnp.float32),
        mesh=mesh,
        compiler_params=pltpu.CompilerParams(
            needs_layout_passes=False, use_tc_tiling_on_sc=False),
        scratch_types=[
            pltpu.VMEM((K, D), jnp.bfloat16),
            pltpu.VMEM((D,), jnp.bfloat16),
            pltpu.VMEM((L,), jnp.float32),
            pltpu.VMEM((L, L), jnp.float32),
            pltpu.VMEM((5, E), jnp.int32),
            pltpu.VMEM((5, E), jnp.int32),
            pltpu.VMEM((5, E), jnp.int32),
            pltpu.VMEM((5, E), jnp.int32),
            pltpu.VMEM((E, D), jnp.bfloat16),
            pltpu.VMEM((E, D), jnp.bfloat16),
            pltpu.VMEM((E, D), jnp.bfloat16),
            pltpu.VMEM((E, D), jnp.bfloat16),
            pltpu.VMEM((E,), jnp.float32),
            pltpu.VMEM((E,), jnp.float32),
            pltpu.SemaphoreType.DMA,
            pltpu.SemaphoreType.DMA,
            pltpu.SemaphoreType.DMA,
            pltpu.SemaphoreType.DMA,
            pltpu.SemaphoreType.DMA,
            pltpu.SemaphoreType.DMA,
            pltpu.SemaphoreType.DMA,
            pltpu.SemaphoreType.DMA,
        ],
    )
    def sc_score(u_hbm, v_hbm, side_hbm, w2_hbm, b2_hbm,
                 src_hbm, dst_hbm, t_hbm, i_hbm, c_hbm, s_hbm,
                 side_v, w2_v, b2_v, mat_v, idxA0, idxA1, idxB0, idxB1,
                 u0, u1, v0, v1, s0, s1,
                 semiA0, semiA1, semiB0, semiB1, semg0, semg1, semo0, semo1):
        wid = lax.axis_index("s") * NC + lax.axis_index("c")
        base = wid * per_w
        pltpu.sync_copy(side_hbm, side_v)
        pltpu.sync_copy(w2_hbm, w2_v)
        pltpu.sync_copy(b2_hbm, b2_v)
        w2b = [w2_v[pl.ds(kk * W, W)] for kk in range(D // W)]
        lanes = lax.iota(jnp.int32, L)
        streams = (src_hbm, dst_hbm, t_hbm, i_hbm, c_hbm)

        def start_idx(k, idx_p, semi):
            cbase = base + k * E
            for r in range(5):
                pltpu.async_copy(
                    streams[r].at[pl.ds(cbase, E)], idx_p.at[r], semi)

        def fire_g(idx_p, semi, u_p, v_p, semg):
            for r in range(5):
                pltpu.make_async_copy(
                    streams[r].at[pl.ds(0, E)], idx_p.at[r], semi).wait()
            pltpu.async_copy(u_hbm.at[idx_p.at[0]], u_p, semg)
            pltpu.async_copy(v_hbm.at[idx_p.at[1]], v_p, semg)

        def drain_g(u_p, v_p, semg):
            pltpu.make_async_copy(u_hbm.at[pl.ds(0, E)], u_p, semg).wait()
            pltpu.make_async_copy(v_hbm.at[pl.ds(0, E)], v_p, semg).wait()

        def drain_o(s_p, semo):
            pltpu.make_async_copy(s_hbm.at[pl.ds(0, E)], s_p, semo).wait()

        def score_chunk(idx_p, u_p, v_p, s_p):
            def gbody(g, carry):
                tvec = idx_p[2, pl.ds(g * L, L)]
                ivec = idx_p[3, pl.ds(g * L, L)] + NT
                cvec = idx_p[4, pl.ds(g * L, L)] + (NT + NI)
                for j in range(L):
                    e = g * L + j
                    t = tvec[j]
                    i = ivec[j]
                    c = cvec[j]
                    parts = []
                    for kk in range(D // W):
                        s = pl.ds(kk * W, W)
                        a = u_p[e, s] + v_p[e, s]
                        b = side_v[t, s] + side_v[i, s]
                        z = (a + b) + side_v[c, s]
                        parts.append(jnp.maximum(z, jnp.bfloat16(0)) * w2b[kk])
                    psum = (parts[0] + parts[1]) + (parts[2] + parts[3])
                    pa, pb = plsc.unpack(
                        psum, format=plsc.PackFormat.INTERLEAVED,
                        preferred_element_type=jnp.float32)
                    mat_v[j, :] = pa + pb
                acc = b2_v[...] + plsc.load_gather(
                    mat_v, [lanes, jnp.full((L,), 0, jnp.int32)])
                for col in range(1, L):
                    acc = acc + plsc.load_gather(
                        mat_v, [lanes, jnp.full((L,), col, jnp.int32)])
                s_p[pl.ds(g * L, L)] = acc
                return carry
            lax.fori_loop(0, E // L, gbody, 0)

        def out(s_p, k, semo):
            pltpu.async_copy(s_p, s_hbm.at[pl.ds(base + k * E, E)], semo)

        QM = (chunks - 1) // 4
        start_idx(0, idxA0, semiA0)
        start_idx(1, idxA1, semiA1)
        start_idx(2, idxB0, semiB0)
        fire_g(idxA0, semiA0, u0, v0, semg0)
        fire_g(idxA1, semiA1, u1, v1, semg1)

        def quad_body(m, carry):
            q = 4 * m
            start_idx(q + 3, idxB1, semiB1)

            @pl.when(m > 0)
            def _():
                drain_o(s0, semo0)

            drain_g(u0, v0, semg0)
            score_chunk(idxA0, u0, v0, s0)
            out(s0, q, semo0)
            fire_g(idxB0, semiB0, u0, v0, semg0)
            start_idx(q + 4, idxA0, semiA0)

            @pl.when(m > 0)
            def _():
                drain_o(s1, semo1)

            drain_g(u1, v1, semg1)
            score_chunk(idxA1, u1, v1, s1)
            out(s1, q + 1, semo1)
            fire_g(idxB1, semiB1, u1, v1, semg1)

            @pl.when(m < QM - 1)
            def _():
                start_idx(q + 5, idxA1, semiA1)

            drain_o(s0, semo0)
            drain_g(u0, v0, semg0)
            score_chunk(idxB0, u0, v0, s0)
            out(s0, q + 2, semo0)
            fire_g(idxA0, semiA0, u0, v0, semg0)

            @pl.when(m < QM - 1)
            def _():
                start_idx(q + 6, idxB0, semiB0)

            drain_o(s1, semo1)
            drain_g(u1, v1, semg1)
            score_chunk(idxB1, u1, v1, s1)
            out(s1, q + 3, semo1)

            @pl.when(m < QM - 1)
            def _():
                fire_g(idxA1, semiA1, u1, v1, semg1)

            return carry

        assert chunks % 4 == 1
        lax.fori_loop(0, QM, quad_body, 0)
        drain_o(s0, semo0)
        drain_g(u0, v0, semg0)
        score_chunk(idxA0, u0, v0, s0)
        out(s0, chunks - 1, semo0)
        drain_o(s0, semo0)
        drain_o(s1, semo1)

    return sc_score


def kernel(hidden, times_table, interval_table, connection_table, W1, b1, W2,
           b2, source, destination, times, intervals, connection_types):
    N, D = hidden.shape
    TH = times_table.shape[1]
    IH = interval_table.shape[1]
    B = source.shape[0]
    NT = times_table.shape[0]
    NI = interval_table.shape[0]
    NCN = connection_table.shape[0]
    E = 80
    assert B % (NW * E) == 0

    U, V = _premul_nodes(hidden, W1)
    K = NT + NI + NCN + ((-(NT + NI + NCN)) % 8)
    sidetab = _premul_side(times_table, interval_table, connection_table,
                           W1, b1, K)
    sc_score = _make_sc_score(B, D, K, E, NT, NI)
    scores = sc_score(U, V, sidetab,
                      W2.reshape(D).astype(jnp.bfloat16),
                      jnp.broadcast_to(b2, (L,)),
                      source, destination, times, intervals, connection_types)
    return scores
